# baseline (speedup 1.0000x reference)
import functools

import jax
import jax.numpy as jnp
from jax import lax
from jax.experimental import pallas as pl
from jax.experimental.pallas import tpu as pltpu
from jax.experimental.pallas import tpu_sc as plsc

G, T, H, E, C = 2, 2048, 2048, 16, 128
TBLK_A = 1024
TBLK_C = 512
ROWS_PER_TILE = G * T * E // 32


def _probs_body(x_ref, w_ref, b_ref, probs_ref, z_ref):
    g = pl.program_id(0)
    tb = pl.program_id(1)
    x = x_ref[0]
    w = w_ref[...]
    b = b_ref[...]
    logits = jax.lax.dot_general(
        w, x, dimension_numbers=(((0,), (1,)), ((), ())),
        preferred_element_type=jnp.float32)
    logits = logits + b.reshape(E, 1)
    m = jnp.max(logits, axis=0, keepdims=True)
    ex = jnp.exp(logits - m)
    s = jnp.sum(ex, axis=0, keepdims=True)
    probs_ref[0] = ex / s
    lse = m + jnp.log(s)
    zpart = jnp.sum(lse * lse).reshape(1, 1)

    @pl.when(jnp.logical_and(g == 0, tb == 0))
    def _():
        z_ref[...] = jnp.zeros_like(z_ref)

    z_ref[...] += zpart


def _first(av, ai, bv, bi):
    return (av > bv) | ((av == bv) & (ai < bi))


def _cex(v, i, islow, j, keepmask):
    pv = jnp.where(islow, jnp.roll(v, -j, 1), jnp.roll(v, j, 1))
    pi = jnp.where(islow, jnp.roll(i, -j, 1), jnp.roll(i, j, 1))
    sf = _first(v, i, pv, pi)
    keep = sf == keepmask
    return jnp.where(keep, v, pv), jnp.where(keep, i, pi)


def _topk_body(p_ref, ei_ref, eg_ref):
    rows = G * E
    v = p_ref[...]
    lane = jax.lax.broadcasted_iota(jnp.int32, (rows, T), 1)
    i = lane
    want = lane < (T // 2)
    islow_by_j = {j: (lane & j) == 0 for j in (1, 2, 4, 8, 16, 32, 64)}
    for k in (2, 4, 8, 16, 32, 64, 128):
        desc = want if k == 128 else want ^ ((lane & k) != 0)
        j = k // 2
        while j >= 1:
            islow = islow_by_j[j]
            v, i = _cex(v, i, islow, j, islow == desc)
            j //= 2
    w = T
    while w > C:
        h = w // 2
        f = _first(v[:, :h], i[:, :h], v[:, h:w], i[:, h:w])
        v = jnp.where(f, v[:, :h], v[:, h:w])
        i = jnp.where(f, i[:, :h], i[:, h:w])
        desc_h = lane[:, :h] < max(h // 2, C)
        for j in (64, 32, 16, 8, 4, 2, 1):
            islow = islow_by_j[j][:, :h]
            v, i = _cex(v, i, islow, j, islow == desc_h)
        w = h
    ei_ref[...] = i
    eg_ref[...] = v


def _disp_body(ei_ref, disp_ref):
    tb = pl.program_id(1)
    t0 = tb * TBLK_C
    ti = jax.lax.broadcasted_iota(jnp.int32, (TBLK_C, E, C), 0) + t0
    hit = ei_ref[0][None, :, :] == ti
    disp_ref[0] = jnp.where(hit, 1.0, 0.0).astype(jnp.float32)


def _comb_body(ei_hbm, eg_hbm, comb_hbm, ei_v, eg_v, rows_v, gbuf, *rest):
    nb = G * T * E // 32 // C
    idx_refs = rest[:nb]
    gidx_v = rest[nb]
    sems = rest[nb + 1:nb + 1 + nb]
    gsem = rest[-1]
    wid = lax.axis_index("s") * 2 + lax.axis_index("c")
    g = wid // E
    e = wid % E

    pltpu.sync_copy(ei_hbm.at[wid], ei_v)
    pltpu.sync_copy(eg_hbm.at[wid], eg_v)

    def zero_row(r, _):
        for cc in range(C // 16):
            rows_v[r, pl.ds(cc * 16, 16)] = jnp.zeros((16,), jnp.float32)
        return 0

    lax.fori_loop(0, C, zero_row, 0)

    iota16 = lax.broadcasted_iota(jnp.int32, (16,), 0)
    zeros16 = jnp.zeros((16,), jnp.float32)
    base = g * (T * E) + e
    for cc in range(C // 16):
        gchunk = eg_v[pl.ds(cc * 16, 16)]
        for r_loc in range(16):
            r = cc * 16 + r_loc
            for col_cc in range(C // 16):
                if col_cc == cc:
                    gbuf[r, pl.ds(col_cc * 16, 16)] = jnp.where(
                        iota16 == r_loc, gchunk, 0.0)
                else:
                    gbuf[r, pl.ds(col_cc * 16, 16)] = zeros16
        tchunk = ei_v[pl.ds(cc * 16, 16)]
        gidx_v[pl.ds(cc * 16, 16)] = base + tchunk * E
    nbatch = ROWS_PER_TILE // C
    for k in range(nbatch):
        for cc in range(C // 16):
            j16 = k * C + cc * 16 + iota16
            idx_refs[k][pl.ds(cc * 16, 16)] = base + j16 * E
    copies = [
        pltpu.async_copy(rows_v, comb_hbm.at[idx_refs[k]], sems[k])
        for k in range(nbatch)
    ]
    for cp in copies:
        cp.wait()
    pltpu.async_copy(gbuf, comb_hbm.at[gidx_v], gsem).wait()
    pltpu.async_copy(gbuf, comb_hbm.at[gidx_v], gsem).wait()


def _make_comb():
    mesh = plsc.VectorSubcoreMesh(core_axis_name="c", subcore_axis_name="s")
    return pl.kernel(
        _comb_body,
        out_type=jax.ShapeDtypeStruct((G * T * E, C), jnp.float32),
        mesh=mesh,
        scratch_types=(
            [
                pltpu.VMEM((C,), jnp.int32),
                pltpu.VMEM((C,), jnp.float32),
                pltpu.VMEM((C, C), jnp.float32),
                pltpu.VMEM((C, C), jnp.float32),
            ]
            + [pltpu.VMEM((C,), jnp.int32)
               for _ in range(ROWS_PER_TILE // C)]
            + [pltpu.VMEM((C,), jnp.int32)]
            + [pltpu.SemaphoreType.DMA for _ in range(ROWS_PER_TILE // C + 1)]
        ),
    )


@functools.partial(jax.jit, static_argnums=())
def _run(x, w, b):
    probs_t, zsum = pl.pallas_call(
        _probs_body,
        grid=(G, T // TBLK_A),
        in_specs=[
            pl.BlockSpec((1, TBLK_A, H), lambda g, tb: (g, tb, 0)),
            pl.BlockSpec((H, E), lambda g, tb: (0, 0)),
            pl.BlockSpec((1, E), lambda g, tb: (0, 0)),
        ],
        out_specs=[
            pl.BlockSpec((1, E, TBLK_A), lambda g, tb: (g, 0, tb)),
            pl.BlockSpec((1, 1), lambda g, tb: (0, 0)),
        ],
        out_shape=[
            jax.ShapeDtypeStruct((G, E, T), jnp.float32),
            jax.ShapeDtypeStruct((1, 1), jnp.float32),
        ],
    )(x, w, b.reshape(1, E))

    ei, eg = pl.pallas_call(
        _topk_body,
        in_specs=[pl.BlockSpec((G * E, T), lambda: (0, 0))],
        out_specs=[
            pl.BlockSpec((G * E, C), lambda: (0, 0)),
            pl.BlockSpec((G * E, C), lambda: (0, 0)),
        ],
        out_shape=[
            jax.ShapeDtypeStruct((G * E, C), jnp.int32),
            jax.ShapeDtypeStruct((G * E, C), jnp.float32),
        ],
    )(probs_t.reshape(G * E, T))

    disp = pl.pallas_call(
        _disp_body,
        grid=(G, T // TBLK_C),
        in_specs=[pl.BlockSpec((1, E, C), lambda g, tb: (g, 0, 0))],
        out_specs=pl.BlockSpec((1, TBLK_C, E, C), lambda g, tb: (g, tb, 0, 0)),
        out_shape=jax.ShapeDtypeStruct((G, T, E, C), jnp.float32),
    )(ei.reshape(G, E, C))

    comb = _make_comb()(ei, eg).reshape(G, T, E, C)

    z_loss = zsum[0, 0] / (G * T)
    return disp, comb, z_loss


def kernel(inputs, kernel, bias, expert_capacity):
    del expert_capacity
    return _run(inputs, kernel, bias)


# speedup vs baseline: 1.2722x; 1.2722x over previous
import functools

import jax
import jax.numpy as jnp
from jax.experimental import pallas as pl

G, T, H, E, C = 2, 2048, 2048, 16, 128
TBLK_A = 1024
TBLK_C = 512


def _probs_body(x_ref, w_ref, b_ref, probs_ref, z_ref):
    g = pl.program_id(0)
    tb = pl.program_id(1)
    x = x_ref[0]
    w = w_ref[...]
    b = b_ref[...]
    logits = jax.lax.dot_general(
        w, x, dimension_numbers=(((0,), (1,)), ((), ())),
        preferred_element_type=jnp.float32)
    logits = logits + b.reshape(E, 1)
    m = jnp.max(logits, axis=0, keepdims=True)
    ex = jnp.exp(logits - m)
    s = jnp.sum(ex, axis=0, keepdims=True)
    probs_ref[0] = ex / s
    lse = m + jnp.log(s)
    zpart = jnp.sum(lse * lse).reshape(1, 1)

    @pl.when(jnp.logical_and(g == 0, tb == 0))
    def _():
        z_ref[...] = jnp.zeros_like(z_ref)

    z_ref[...] += zpart


def _first(av, ai, bv, bi):
    return (av > bv) | ((av == bv) & (ai < bi))


def _cex(v, i, islow, j, keepmask):
    pv = jnp.where(islow, jnp.roll(v, -j, 1), jnp.roll(v, j, 1))
    pi = jnp.where(islow, jnp.roll(i, -j, 1), jnp.roll(i, j, 1))
    sf = _first(v, i, pv, pi)
    keep = sf == keepmask
    return jnp.where(keep, v, pv), jnp.where(keep, i, pi)


def _topk_body(p_ref, ei_ref, eg_ref):
    rows = G * E
    v = p_ref[...]
    lane = jax.lax.broadcasted_iota(jnp.int32, (rows, T), 1)
    i = lane
    want = lane < (T // 2)
    islow_by_j = {j: (lane & j) == 0 for j in (1, 2, 4, 8, 16, 32, 64)}
    for k in (2, 4, 8, 16, 32, 64, 128):
        desc = want if k == 128 else want ^ ((lane & k) != 0)
        j = k // 2
        while j >= 1:
            islow = islow_by_j[j]
            v, i = _cex(v, i, islow, j, islow == desc)
            j //= 2
    w = T
    while w > C:
        h = w // 2
        f = _first(v[:, :h], i[:, :h], v[:, h:w], i[:, h:w])
        v = jnp.where(f, v[:, :h], v[:, h:w])
        i = jnp.where(f, i[:, :h], i[:, h:w])
        desc_h = lane[:, :h] < max(h // 2, C)
        for j in (64, 32, 16, 8, 4, 2, 1):
            islow = islow_by_j[j][:, :h]
            v, i = _cex(v, i, islow, j, islow == desc_h)
        w = h
    ei_ref[...] = i
    eg_ref[...] = v


def _mask_body(ei_ref, eg_ref, disp_ref, comb_ref):
    tb = pl.program_id(1)
    t0 = tb * TBLK_C
    ti = jax.lax.broadcasted_iota(jnp.int32, (TBLK_C, E, C), 0) + t0
    hit = ei_ref[0][None, :, :] == ti
    disp_ref[0] = jnp.where(hit, 1.0, 0.0).astype(jnp.float32)
    comb_ref[0] = jnp.where(hit, eg_ref[0][None, :, :], 0.0).astype(jnp.float32)


@functools.partial(jax.jit, static_argnums=())
def _run(x, w, b):
    probs_t, zsum = pl.pallas_call(
        _probs_body,
        grid=(G, T // TBLK_A),
        in_specs=[
            pl.BlockSpec((1, TBLK_A, H), lambda g, tb: (g, tb, 0)),
            pl.BlockSpec((H, E), lambda g, tb: (0, 0)),
            pl.BlockSpec((1, E), lambda g, tb: (0, 0)),
        ],
        out_specs=[
            pl.BlockSpec((1, E, TBLK_A), lambda g, tb: (g, 0, tb)),
            pl.BlockSpec((1, 1), lambda g, tb: (0, 0)),
        ],
        out_shape=[
            jax.ShapeDtypeStruct((G, E, T), jnp.float32),
            jax.ShapeDtypeStruct((1, 1), jnp.float32),
        ],
    )(x, w, b.reshape(1, E))

    ei, eg = pl.pallas_call(
        _topk_body,
        in_specs=[pl.BlockSpec((G * E, T), lambda: (0, 0))],
        out_specs=[
            pl.BlockSpec((G * E, C), lambda: (0, 0)),
            pl.BlockSpec((G * E, C), lambda: (0, 0)),
        ],
        out_shape=[
            jax.ShapeDtypeStruct((G * E, C), jnp.int32),
            jax.ShapeDtypeStruct((G * E, C), jnp.float32),
        ],
    )(probs_t.reshape(G * E, T))

    disp, comb = pl.pallas_call(
        _mask_body,
        grid=(G, T // TBLK_C),
        in_specs=[
            pl.BlockSpec((1, E, C), lambda g, tb: (g, 0, 0)),
            pl.BlockSpec((1, E, C), lambda g, tb: (g, 0, 0)),
        ],
        out_specs=[
            pl.BlockSpec((1, TBLK_C, E, C), lambda g, tb: (g, tb, 0, 0)),
            pl.BlockSpec((1, TBLK_C, E, C), lambda g, tb: (g, tb, 0, 0)),
        ],
        out_shape=[
            jax.ShapeDtypeStruct((G, T, E, C), jnp.float32),
            jax.ShapeDtypeStruct((G, T, E, C), jnp.float32),
        ],
    )(ei.reshape(G, E, C), eg.reshape(G, E, C))

    z_loss = zsum[0, 0] / (G * T)
    return disp, comb, z_loss


def kernel(inputs, kernel, bias, expert_capacity):
    del expert_capacity
    return _run(inputs, kernel, bias)


# speedup vs baseline: 1.2819x; 1.0076x over previous
import functools

import jax
import jax.numpy as jnp
from jax.experimental import pallas as pl

G, T, H, E, C = 2, 2048, 2048, 16, 128
TBLK_A = 1024
TBLK_C = 1024


def _probs_body(x_ref, w_ref, b_ref, probs_ref, z_ref):
    g = pl.program_id(0)
    tb = pl.program_id(1)
    x = x_ref[0]
    w = w_ref[...]
    b = b_ref[...]
    logits = jax.lax.dot_general(
        w, x, dimension_numbers=(((0,), (1,)), ((), ())),
        preferred_element_type=jnp.float32)
    logits = logits + b.reshape(E, 1)
    m = jnp.max(logits, axis=0, keepdims=True)
    ex = jnp.exp(logits - m)
    s = jnp.sum(ex, axis=0, keepdims=True)
    probs_ref[0] = ex / s
    lse = m + jnp.log(s)
    zpart = jnp.sum(lse * lse).reshape(1, 1)

    @pl.when(jnp.logical_and(g == 0, tb == 0))
    def _():
        z_ref[...] = jnp.zeros_like(z_ref)

    z_ref[...] += zpart


def _first(av, ai, bv, bi):
    return (av > bv) | ((av == bv) & (ai < bi))


def _cex(v, i, islow, j, keepmask):
    pv = jnp.where(islow, jnp.roll(v, -j, 1), jnp.roll(v, j, 1))
    pi = jnp.where(islow, jnp.roll(i, -j, 1), jnp.roll(i, j, 1))
    sf = _first(v, i, pv, pi)
    keep = sf == keepmask
    return jnp.where(keep, v, pv), jnp.where(keep, i, pi)


def _topk_body(p_ref, ei_ref, eg_ref):
    rows = G * E
    v = p_ref[...]
    lane = jax.lax.broadcasted_iota(jnp.int32, (rows, T), 1)
    i = lane
    want = lane < (T // 2)
    islow_by_j = {j: (lane & j) == 0 for j in (1, 2, 4, 8, 16, 32, 64)}
    for k in (2, 4, 8, 16, 32, 64, 128):
        desc = want if k == 128 else want ^ ((lane & k) != 0)
        j = k // 2
        while j >= 1:
            islow = islow_by_j[j]
            v, i = _cex(v, i, islow, j, islow == desc)
            j //= 2
    w = T
    while w > C:
        h = w // 2
        f = _first(v[:, :h], i[:, :h], v[:, h:w], i[:, h:w])
        v = jnp.where(f, v[:, :h], v[:, h:w])
        i = jnp.where(f, i[:, :h], i[:, h:w])
        desc_h = lane[:, :h] < max(h // 2, C)
        for j in (64, 32, 16, 8, 4, 2, 1):
            islow = islow_by_j[j][:, :h]
            v, i = _cex(v, i, islow, j, islow == desc_h)
        w = h
    ei_ref[...] = i
    eg_ref[...] = v


def _mask_body(ei_ref, eg_ref, disp_ref, comb_ref):
    tb = pl.program_id(1)
    t0 = tb * TBLK_C
    ti = jax.lax.broadcasted_iota(jnp.int32, (TBLK_C, E, C), 0) + t0
    hit = ei_ref[0][None, :, :] == ti
    disp_ref[0] = jnp.where(hit, 1.0, 0.0).astype(jnp.float32)
    comb_ref[0] = jnp.where(hit, eg_ref[0][None, :, :], 0.0).astype(jnp.float32)


@functools.partial(jax.jit, static_argnums=())
def _run(x, w, b):
    probs_t, zsum = pl.pallas_call(
        _probs_body,
        grid=(G, T // TBLK_A),
        in_specs=[
            pl.BlockSpec((1, TBLK_A, H), lambda g, tb: (g, tb, 0)),
            pl.BlockSpec((H, E), lambda g, tb: (0, 0)),
            pl.BlockSpec((1, E), lambda g, tb: (0, 0)),
        ],
        out_specs=[
            pl.BlockSpec((1, E, TBLK_A), lambda g, tb: (g, 0, tb)),
            pl.BlockSpec((1, 1), lambda g, tb: (0, 0)),
        ],
        out_shape=[
            jax.ShapeDtypeStruct((G, E, T), jnp.float32),
            jax.ShapeDtypeStruct((1, 1), jnp.float32),
        ],
    )(x, w, b.reshape(1, E))

    ei, eg = pl.pallas_call(
        _topk_body,
        in_specs=[pl.BlockSpec((G * E, T), lambda: (0, 0))],
        out_specs=[
            pl.BlockSpec((G * E, C), lambda: (0, 0)),
            pl.BlockSpec((G * E, C), lambda: (0, 0)),
        ],
        out_shape=[
            jax.ShapeDtypeStruct((G * E, C), jnp.int32),
            jax.ShapeDtypeStruct((G * E, C), jnp.float32),
        ],
    )(probs_t.reshape(G * E, T))

    disp, comb = pl.pallas_call(
        _mask_body,
        grid=(G, T // TBLK_C),
        in_specs=[
            pl.BlockSpec((1, E, C), lambda g, tb: (g, 0, 0)),
            pl.BlockSpec((1, E, C), lambda g, tb: (g, 0, 0)),
        ],
        out_specs=[
            pl.BlockSpec((1, TBLK_C, E, C), lambda g, tb: (g, tb, 0, 0)),
            pl.BlockSpec((1, TBLK_C, E, C), lambda g, tb: (g, tb, 0, 0)),
        ],
        out_shape=[
            jax.ShapeDtypeStruct((G, T, E, C), jnp.float32),
            jax.ShapeDtypeStruct((G, T, E, C), jnp.float32),
        ],
    )(ei.reshape(G, E, C), eg.reshape(G, E, C))

    z_loss = zsum[0, 0] / (G * T)
    return disp, comb, z_loss


def kernel(inputs, kernel, bias, expert_capacity):
    del expert_capacity
    return _run(inputs, kernel, bias)
